# baseline (speedup 1.0000x reference)
import functools

import jax
import jax.numpy as jnp
from jax import lax
from jax.experimental import pallas as pl
from jax.experimental.pallas import tpu as pltpu
from jax.experimental.pallas import tpu_sc as plsc

D_MODEL = 1024
B_TOTAL = 4 * 4096
NC, NS = 2, 16
NW = NC * NS
B_PER_W = B_TOTAL // NW
CHUNK = 16
NCHUNK = B_PER_W // CHUNK
NBUF = 4
LANES = 16
SCALE = 32.0

_mesh = plsc.VectorSubcoreMesh(core_axis_name="c", subcore_axis_name="s")


@functools.partial(
    pl.kernel,
    out_type=jax.ShapeDtypeStruct((B_TOTAL, D_MODEL), jnp.float32),
    mesh=_mesh,
    scratch_types=[
        pltpu.VMEM((NCHUNK, CHUNK), jnp.int32),
        pltpu.VMEM((NBUF, CHUNK, D_MODEL), jnp.float32),
        pltpu.SemaphoreType.DMA,
        pltpu.SemaphoreType.DMA,
        pltpu.SemaphoreType.DMA,
        pltpu.SemaphoreType.DMA,
        pltpu.SemaphoreType.DMA,
        pltpu.SemaphoreType.DMA,
        pltpu.SemaphoreType.DMA,
        pltpu.SemaphoreType.DMA,
    ],
)
def _emb_kernel(idx_hbm, table_hbm, out_hbm, idx_v, ring,
                g0, g1, g2, g3, s0, s1, s2, s3):
    wid = lax.axis_index("s") * NC + lax.axis_index("c")
    pltpu.sync_copy(idx_hbm.at[wid], idx_v)
    base = wid * B_PER_W
    gsems = (g0, g1, g2, g3)
    ssems = (s0, s1, s2, s3)

    def start_gather(c, b):
        return pltpu.async_copy(table_hbm.at[idx_v.at[c]], ring.at[b],
                                gsems[b])

    def wait_gather(b):
        pltpu.make_async_copy(table_hbm.at[idx_v.at[0]], ring.at[b],
                              gsems[b]).wait()

    def start_store(c, b):
        return pltpu.async_copy(
            ring.at[b], out_hbm.at[pl.ds(base + c * CHUNK, CHUNK)], ssems[b])

    def wait_store(b):
        pltpu.make_async_copy(
            ring.at[b], out_hbm.at[pl.ds(base, CHUNK)], ssems[b]).wait()

    for b in range(NBUF):
        start_gather(b, b)

    @pl.loop(0, NCHUNK, step=NBUF)
    def _group(g):
        for b in range(NBUF):
            c = g + b
            wait_gather(b)

            @pl.loop(0, CHUNK)
            def _scale(r, b=b):
                for v in range(D_MODEL // LANES):
                    sl = pl.ds(v * LANES, LANES)
                    ring[b, r, sl] = ring[b, r, sl] * SCALE

            start_store(c, b)
        for b in range(NBUF):
            c = g + b

            @pl.when(c + NBUF < NCHUNK)
            def _refill(c=c, b=b):
                wait_store(b)
                start_gather(c + NBUF, b)

    for b in range(NBUF):
        wait_store(b)


def kernel(x, table):
    idx = x.reshape(NW, NCHUNK, CHUNK)
    out = _emb_kernel(idx, table)
    return out.reshape(x.shape[0], x.shape[1], D_MODEL)
